# baseline (speedup 1.0000x reference)
import functools

import jax
import jax.numpy as jnp
from jax import lax
from jax.experimental import pallas as pl
from jax.experimental.pallas import tpu as pltpu
from jax.experimental.pallas import tpu_sc as plsc

N_NODES = 10000
N_EDGES = 320000
D = 128
NC = 2
NS = 16
NW = NC * NS
EPW = N_EDGES // NW
CH = 80
NCH = EPW // CH
DEPTH = 4
NPAD = 10240
RPT = NPAD // NS
def _sc_agg_body(x_hbm, src_hbm, dst_hbm, zf_hbm, zd_hbm,
                 part_hbm, deg_hbm,
                 srcr, dstr, rows, ones_v, acc_sh, deg_sh,
                 gsem, ssem, isem,
                 *, with_deg):
    cid = lax.axis_index("c")
    sid = lax.axis_index("s")
    wid = cid * NS + sid
    base = wid * EPW

    for c in range(DEPTH - 1):
        pltpu.sync_copy(src_hbm.at[pl.ds(base + c * CH, CH)], srcr.at[c])
        pltpu.sync_copy(dst_hbm.at[pl.ds(base + c * CH, CH)], dstr.at[c])
        pltpu.async_copy(x_hbm.at[srcr.at[c]], rows.at[c], gsem)
    _off3 = (DEPTH - 1) * CH
    pltpu.async_copy(src_hbm.at[pl.ds(base + _off3, CH)],
                     srcr.at[DEPTH - 1], isem)
    pltpu.async_copy(dst_hbm.at[pl.ds(base + _off3, CH)],
                     dstr.at[DEPTH - 1], isem)

    pltpu.sync_copy(zf_hbm.at[pl.ds(sid * RPT, RPT)],
                    acc_sh.at[pl.ds(sid * RPT, RPT)])
    if with_deg:
        @pl.when(sid == 0)
        def _():
            pltpu.sync_copy(zd_hbm, deg_sh)
        for i in range(CH // 16):
            ones_v[pl.ds(i * 16, 16)] = jnp.ones((16,), jnp.float32)
    plsc.subcore_barrier()

    def step(j, carry):
        s = lax.rem(j, DEPTH)
        s3 = lax.rem(j + DEPTH - 1, DEPTH)
        pltpu.make_async_copy(x_hbm.at[srcr.at[s]], rows.at[s], gsem).wait()
        @pl.when(j > 0)
        def _():
            pltpu.make_async_copy(rows.at[s3], acc_sh.at[dstr.at[s3]],
                                  ssem).wait()
            if with_deg:
                pltpu.make_async_copy(ones_v, deg_sh.at[dstr.at[s3]],
                                      ssem).wait()
        @pl.when(j + DEPTH - 1 < NCH)
        def _():
            pltpu.make_async_copy(src_hbm.at[pl.ds(base, CH)], srcr.at[s3],
                                  isem).wait()
            pltpu.make_async_copy(dst_hbm.at[pl.ds(base, CH)], dstr.at[s3],
                                  isem).wait()
            pltpu.async_copy(x_hbm.at[srcr.at[s3]], rows.at[s3], gsem)
        @pl.when(j + DEPTH < NCH)
        def _():
            off = base + (j + DEPTH) * CH
            pltpu.async_copy(src_hbm.at[pl.ds(off, CH)], srcr.at[s], isem)
            pltpu.async_copy(dst_hbm.at[pl.ds(off, CH)], dstr.at[s], isem)
        pltpu.async_copy(rows.at[s], acc_sh.at[dstr.at[s]], ssem, add=True)
        if with_deg:
            pltpu.async_copy(ones_v, deg_sh.at[dstr.at[s]], ssem, add=True)
        return carry

    lax.fori_loop(0, NCH, step, 0)
    _last = (NCH - 1) % DEPTH
    pltpu.make_async_copy(rows.at[_last], acc_sh.at[dstr.at[_last]],
                          ssem).wait()
    if with_deg:
        pltpu.make_async_copy(ones_v, deg_sh.at[dstr.at[_last]], ssem).wait()
    plsc.subcore_barrier()

    pltpu.sync_copy(acc_sh.at[pl.ds(sid * RPT, RPT)],
                    part_hbm.at[cid, pl.ds(sid * RPT, RPT)])
    if with_deg:
        @pl.when(sid == 0)
        def _():
            pltpu.sync_copy(deg_sh, deg_hbm.at[cid])


def _make_sc_agg(with_deg):
    mesh = plsc.VectorSubcoreMesh(core_axis_name="c", subcore_axis_name="s")
    return functools.partial(
        pl.kernel,
        mesh=mesh,
        out_type=[
            jax.ShapeDtypeStruct((NC, NPAD, D), jnp.float32),
            jax.ShapeDtypeStruct((NC, NPAD), jnp.float32),
        ],
        scratch_types=[
            pltpu.VMEM((DEPTH, CH), jnp.int32),
            pltpu.VMEM((DEPTH, CH), jnp.int32),
            pltpu.VMEM((DEPTH, CH, D), jnp.float32),
            pltpu.VMEM((CH,), jnp.float32),
            pltpu.VMEM_SHARED((NPAD, D), jnp.float32),
            pltpu.VMEM_SHARED((NPAD,), jnp.float32),
            pltpu.SemaphoreType.DMA,
            pltpu.SemaphoreType.DMA,
            pltpu.SemaphoreType.DMA,
        ],
    )(functools.partial(_sc_agg_body, with_deg=with_deg))


_sc_agg_deg = _make_sc_agg(True)
_sc_agg_nodeg = _make_sc_agg(False)


def _dense_body(p_ref, deg_ref, x_ref, wl_ref, wr_ref, b_ref, o_ref, *, relu):
    deg = jnp.maximum(deg_ref[0] + deg_ref[1], 1.0)
    agg = (p_ref[0] + p_ref[1]) / deg
    dn = (((1,), (1,)), ((), ()))
    out = (lax.dot_general(agg, wl_ref[...], dn,
                           preferred_element_type=jnp.float32)
           + lax.dot_general(x_ref[...], wr_ref[...], dn,
                             preferred_element_type=jnp.float32)
           + b_ref[...])
    o_ref[...] = jnp.maximum(out, 0.0) if relu else out


def _dense(parts, deg3, xin, wlT, wrT, b, relu):
    BM = 2000
    grid = (N_NODES // BM,)
    return pl.pallas_call(
        functools.partial(_dense_body, relu=relu),
        grid=grid,
        in_specs=[
            pl.BlockSpec((NC, BM, D), lambda i: (0, i, 0)),
            pl.BlockSpec((NC, BM, 1), lambda i: (0, i, 0)),
            pl.BlockSpec((BM, D), lambda i: (i, 0)),
            pl.BlockSpec((D, D), lambda i: (0, 0)),
            pl.BlockSpec((D, D), lambda i: (0, 0)),
            pl.BlockSpec((1, D), lambda i: (0, 0)),
        ],
        out_specs=pl.BlockSpec((BM, D), lambda i: (i, 0)),
        out_shape=jax.ShapeDtypeStruct((N_NODES, D), jnp.float32),
    )(parts, deg3, xin, wlT, wrT, b)


def kernel(x, edge_index, W1l, b1l, W1r, W2l, b2l, W2r):
    src = edge_index[0].astype(jnp.int32)
    dst = edge_index[1].astype(jnp.int32)
    zf = jnp.zeros((NPAD, D), jnp.float32)
    zd = jnp.zeros((NPAD,), jnp.float32)

    part1, deg = _sc_agg_deg(x, src, dst, zf, zd)
    deg3 = deg.reshape(NC, NPAD, 1)
    h = _dense(part1, deg3, x, W1l, W1r, b1l.reshape(1, D), relu=True)
    part2, _ = _sc_agg_nodeg(h, src, dst, zf, zd)
    out = _dense(part2, deg3, h, W2l, W2r, b2l.reshape(1, D), relu=False)
    return out


# speedup vs baseline: 1.0005x; 1.0005x over previous
import functools

import jax
import jax.numpy as jnp
from jax import lax
from jax.experimental import pallas as pl
from jax.experimental.pallas import tpu as pltpu
from jax.experimental.pallas import tpu_sc as plsc

N_NODES = 10000
N_EDGES = 320000
D = 128
NC = 2
NS = 16
NW = NC * NS
EPW = N_EDGES // NW
CH = 80
NCH = EPW // CH
DEPTH = 4
IDXD = DEPTH + 1
NPAD = 10240
RPT = NPAD // NS
def _sc_agg_body(x_hbm, src_hbm, dst_hbm, zf_hbm, zd_hbm,
                 part_hbm, deg_hbm,
                 srcr, dstr, rows, ones_v, acc_sh, deg_sh,
                 gsem, ssem, isem,
                 *, with_deg):
    cid = lax.axis_index("c")
    sid = lax.axis_index("s")
    wid = cid * NS + sid
    base = wid * EPW

    for c in range(DEPTH - 1):
        pltpu.sync_copy(src_hbm.at[pl.ds(base + c * CH, CH)], srcr.at[c])
        pltpu.sync_copy(dst_hbm.at[pl.ds(base + c * CH, CH)], dstr.at[c])
        pltpu.async_copy(x_hbm.at[srcr.at[c]], rows.at[c], gsem)
    _off3 = (DEPTH - 1) * CH
    pltpu.async_copy(src_hbm.at[pl.ds(base + _off3, CH)],
                     srcr.at[DEPTH - 1], isem)
    pltpu.async_copy(dst_hbm.at[pl.ds(base + _off3, CH)],
                     dstr.at[DEPTH - 1], isem)

    pltpu.sync_copy(zf_hbm.at[pl.ds(sid * RPT, RPT)],
                    acc_sh.at[pl.ds(sid * RPT, RPT)])
    if with_deg:
        @pl.when(sid == 0)
        def _():
            pltpu.sync_copy(zd_hbm, deg_sh)
        for i in range(CH // 16):
            ones_v[pl.ds(i * 16, 16)] = jnp.ones((16,), jnp.float32)
    plsc.subcore_barrier()

    def step(j, carry):
        s = lax.rem(j, DEPTH)
        s3 = lax.rem(j + DEPTH - 1, DEPTH)
        i0 = lax.rem(j, IDXD)
        i1 = lax.rem(j + IDXD - 1, IDXD)
        i3 = lax.rem(j + DEPTH - 1, IDXD)
        i4 = lax.rem(j + DEPTH, IDXD)
        pltpu.make_async_copy(x_hbm.at[srcr.at[i0]], rows.at[s], gsem).wait()
        @pl.when(j > 0)
        def _():
            pltpu.make_async_copy(rows.at[s3], acc_sh.at[dstr.at[i1]],
                                  ssem).wait()
            if with_deg:
                pltpu.make_async_copy(ones_v, deg_sh.at[dstr.at[i1]],
                                      ssem).wait()
        @pl.when(j + DEPTH - 1 < NCH)
        def _():
            pltpu.make_async_copy(src_hbm.at[pl.ds(base, CH)], srcr.at[i3],
                                  isem).wait()
            pltpu.make_async_copy(dst_hbm.at[pl.ds(base, CH)], dstr.at[i3],
                                  isem).wait()
            pltpu.async_copy(x_hbm.at[srcr.at[i3]], rows.at[s3], gsem)
        @pl.when(j + DEPTH < NCH)
        def _():
            off = base + (j + DEPTH) * CH
            pltpu.async_copy(src_hbm.at[pl.ds(off, CH)], srcr.at[i4], isem)
            pltpu.async_copy(dst_hbm.at[pl.ds(off, CH)], dstr.at[i4], isem)
        pltpu.async_copy(rows.at[s], acc_sh.at[dstr.at[i0]], ssem, add=True)
        if with_deg:
            pltpu.async_copy(ones_v, deg_sh.at[dstr.at[i0]], ssem, add=True)
        return carry

    lax.fori_loop(0, NCH, step, 0)
    _lr = (NCH - 1) % DEPTH
    _li = (NCH - 1) % IDXD
    pltpu.make_async_copy(rows.at[_lr], acc_sh.at[dstr.at[_li]],
                          ssem).wait()
    if with_deg:
        pltpu.make_async_copy(ones_v, deg_sh.at[dstr.at[_li]], ssem).wait()
    plsc.subcore_barrier()

    pltpu.sync_copy(acc_sh.at[pl.ds(sid * RPT, RPT)],
                    part_hbm.at[cid, pl.ds(sid * RPT, RPT)])
    if with_deg:
        @pl.when(sid == 0)
        def _():
            pltpu.sync_copy(deg_sh, deg_hbm.at[cid])


def _make_sc_agg(with_deg):
    mesh = plsc.VectorSubcoreMesh(core_axis_name="c", subcore_axis_name="s")
    return functools.partial(
        pl.kernel,
        mesh=mesh,
        out_type=[
            jax.ShapeDtypeStruct((NC, NPAD, D), jnp.float32),
            jax.ShapeDtypeStruct((NC, NPAD), jnp.float32),
        ],
        scratch_types=[
            pltpu.VMEM((IDXD, CH), jnp.int32),
            pltpu.VMEM((IDXD, CH), jnp.int32),
            pltpu.VMEM((DEPTH, CH, D), jnp.float32),
            pltpu.VMEM((CH,), jnp.float32),
            pltpu.VMEM_SHARED((NPAD, D), jnp.float32),
            pltpu.VMEM_SHARED((NPAD,), jnp.float32),
            pltpu.SemaphoreType.DMA,
            pltpu.SemaphoreType.DMA,
            pltpu.SemaphoreType.DMA,
        ],
    )(functools.partial(_sc_agg_body, with_deg=with_deg))


_sc_agg_deg = _make_sc_agg(True)
_sc_agg_nodeg = _make_sc_agg(False)


def _dense_body(p_ref, deg_ref, x_ref, wl_ref, wr_ref, b_ref, o_ref, *, relu):
    deg = jnp.maximum(deg_ref[0] + deg_ref[1], 1.0)
    agg = (p_ref[0] + p_ref[1]) / deg
    dn = (((1,), (1,)), ((), ()))
    out = (lax.dot_general(agg, wl_ref[...], dn,
                           preferred_element_type=jnp.float32)
           + lax.dot_general(x_ref[...], wr_ref[...], dn,
                             preferred_element_type=jnp.float32)
           + b_ref[...])
    o_ref[...] = jnp.maximum(out, 0.0) if relu else out


def _dense(parts, deg3, xin, wlT, wrT, b, relu):
    BM = 2000
    grid = (N_NODES // BM,)
    return pl.pallas_call(
        functools.partial(_dense_body, relu=relu),
        grid=grid,
        in_specs=[
            pl.BlockSpec((NC, BM, D), lambda i: (0, i, 0)),
            pl.BlockSpec((NC, BM, 1), lambda i: (0, i, 0)),
            pl.BlockSpec((BM, D), lambda i: (i, 0)),
            pl.BlockSpec((D, D), lambda i: (0, 0)),
            pl.BlockSpec((D, D), lambda i: (0, 0)),
            pl.BlockSpec((1, D), lambda i: (0, 0)),
        ],
        out_specs=pl.BlockSpec((BM, D), lambda i: (i, 0)),
        out_shape=jax.ShapeDtypeStruct((N_NODES, D), jnp.float32),
    )(parts, deg3, xin, wlT, wrT, b)


def kernel(x, edge_index, W1l, b1l, W1r, W2l, b2l, W2r):
    src = edge_index[0].astype(jnp.int32)
    dst = edge_index[1].astype(jnp.int32)
    zf = jnp.zeros((NPAD, D), jnp.float32)
    zd = jnp.zeros((NPAD,), jnp.float32)

    part1, deg = _sc_agg_deg(x, src, dst, zf, zd)
    deg3 = deg.reshape(NC, NPAD, 1)
    h = _dense(part1, deg3, x, W1l, W1r, b1l.reshape(1, D), relu=True)
    part2, _ = _sc_agg_nodeg(h, src, dst, zf, zd)
    out = _dense(part2, deg3, h, W2l, W2r, b2l.reshape(1, D), relu=False)
    return out


# speedup vs baseline: 1.0141x; 1.0135x over previous
import functools

import jax
import jax.numpy as jnp
from jax import lax
from jax.experimental import pallas as pl
from jax.experimental.pallas import tpu as pltpu
from jax.experimental.pallas import tpu_sc as plsc

N_NODES = 10000
N_EDGES = 320000
D = 128
NC = 2
NS = 16
NW = NC * NS
EPW = N_EDGES // NW
CH = 80
NCH = EPW // CH
DEPTH = 4
NPAD = 10240
RPT = NPAD // NS
_SHIFT = 14


def _sc_agg_body(x_hbm, pk_hbm, zf_hbm, zd_hbm,
                 part_hbm, deg_hbm,
                 pkr, srcr, dstr, rows, ones_v, acc_sh, deg_sh,
                 gsem, ssem, isem,
                 *, with_deg):
    cid = lax.axis_index("c")
    sid = lax.axis_index("s")
    wid = cid * NS + sid
    base = wid * EPW

    def unpack(slot):
        for k in range(CH // 16):
            pk = pkr[slot, pl.ds(k * 16, 16)]
            srcr[slot, pl.ds(k * 16, 16)] = lax.rem(pk, 1 << _SHIFT)
            dstr[slot, pl.ds(k * 16, 16)] = lax.shift_right_logical(
                pk, _SHIFT)

    for c in range(DEPTH - 1):
        pltpu.sync_copy(pk_hbm.at[pl.ds(base + c * CH, CH)], pkr.at[c])
        unpack(c)
        pltpu.async_copy(x_hbm.at[srcr.at[c]], rows.at[c], gsem)
    pltpu.async_copy(pk_hbm.at[pl.ds(base + (DEPTH - 1) * CH, CH)],
                     pkr.at[DEPTH - 1], isem)

    pltpu.sync_copy(zf_hbm.at[pl.ds(sid * RPT, RPT)],
                    acc_sh.at[pl.ds(sid * RPT, RPT)])
    if with_deg:
        @pl.when(sid == 0)
        def _():
            pltpu.sync_copy(zd_hbm, deg_sh)
        for i in range(CH // 16):
            ones_v[pl.ds(i * 16, 16)] = jnp.ones((16,), jnp.float32)
    plsc.subcore_barrier()

    def step(j, carry):
        s = lax.rem(j, DEPTH)
        s3 = lax.rem(j + DEPTH - 1, DEPTH)
        pltpu.make_async_copy(x_hbm.at[srcr.at[s]], rows.at[s], gsem).wait()
        @pl.when(j > 0)
        def _():
            pltpu.make_async_copy(rows.at[s3], acc_sh.at[dstr.at[s3]],
                                  ssem).wait()
            if with_deg:
                pltpu.make_async_copy(ones_v, deg_sh.at[dstr.at[s3]],
                                      ssem).wait()
        @pl.when(j + DEPTH - 1 < NCH)
        def _():
            pltpu.make_async_copy(pk_hbm.at[pl.ds(base, CH)], pkr.at[s3],
                                  isem).wait()
            unpack(s3)
            pltpu.async_copy(x_hbm.at[srcr.at[s3]], rows.at[s3], gsem)
        @pl.when(j + DEPTH < NCH)
        def _():
            pltpu.async_copy(pk_hbm.at[pl.ds(base + (j + DEPTH) * CH, CH)],
                             pkr.at[s], isem)
        pltpu.async_copy(rows.at[s], acc_sh.at[dstr.at[s]], ssem, add=True)
        if with_deg:
            pltpu.async_copy(ones_v, deg_sh.at[dstr.at[s]], ssem, add=True)
        return carry

    lax.fori_loop(0, NCH, step, 0)
    _last = (NCH - 1) % DEPTH
    pltpu.make_async_copy(rows.at[_last], acc_sh.at[dstr.at[_last]],
                          ssem).wait()
    if with_deg:
        pltpu.make_async_copy(ones_v, deg_sh.at[dstr.at[_last]], ssem).wait()
    plsc.subcore_barrier()

    pltpu.sync_copy(acc_sh.at[pl.ds(sid * RPT, RPT)],
                    part_hbm.at[cid, pl.ds(sid * RPT, RPT)])
    if with_deg:
        @pl.when(sid == 0)
        def _():
            pltpu.sync_copy(deg_sh, deg_hbm.at[cid])


def _make_sc_agg(with_deg):
    mesh = plsc.VectorSubcoreMesh(core_axis_name="c", subcore_axis_name="s")
    return functools.partial(
        pl.kernel,
        mesh=mesh,
        out_type=[
            jax.ShapeDtypeStruct((NC, NPAD, D), jnp.float32),
            jax.ShapeDtypeStruct((NC, NPAD), jnp.float32),
        ],
        scratch_types=[
            pltpu.VMEM((DEPTH, CH), jnp.int32),
            pltpu.VMEM((DEPTH, CH), jnp.int32),
            pltpu.VMEM((DEPTH, CH), jnp.int32),
            pltpu.VMEM((DEPTH, CH, D), jnp.float32),
            pltpu.VMEM((CH,), jnp.float32),
            pltpu.VMEM_SHARED((NPAD, D), jnp.float32),
            pltpu.VMEM_SHARED((NPAD,), jnp.float32),
            pltpu.SemaphoreType.DMA,
            pltpu.SemaphoreType.DMA,
            pltpu.SemaphoreType.DMA,
        ],
    )(functools.partial(_sc_agg_body, with_deg=with_deg))


_sc_agg_deg = _make_sc_agg(True)
_sc_agg_nodeg = _make_sc_agg(False)


def _dense_body(p_ref, deg_ref, x_ref, wl_ref, wr_ref, b_ref, o_ref, *, relu):
    deg = jnp.maximum(deg_ref[0] + deg_ref[1], 1.0)
    agg = (p_ref[0] + p_ref[1]) / deg
    dn = (((1,), (1,)), ((), ()))
    out = (lax.dot_general(agg, wl_ref[...], dn,
                           preferred_element_type=jnp.float32)
           + lax.dot_general(x_ref[...], wr_ref[...], dn,
                             preferred_element_type=jnp.float32)
           + b_ref[...])
    o_ref[...] = jnp.maximum(out, 0.0) if relu else out


def _dense(parts, deg3, xin, wl, wr, b, relu):
    BM = 2000
    grid = (N_NODES // BM,)
    return pl.pallas_call(
        functools.partial(_dense_body, relu=relu),
        grid=grid,
        in_specs=[
            pl.BlockSpec((NC, BM, D), lambda i: (0, i, 0)),
            pl.BlockSpec((NC, BM, 1), lambda i: (0, i, 0)),
            pl.BlockSpec((BM, D), lambda i: (i, 0)),
            pl.BlockSpec((D, D), lambda i: (0, 0)),
            pl.BlockSpec((D, D), lambda i: (0, 0)),
            pl.BlockSpec((1, D), lambda i: (0, 0)),
        ],
        out_specs=pl.BlockSpec((BM, D), lambda i: (i, 0)),
        out_shape=jax.ShapeDtypeStruct((N_NODES, D), jnp.float32),
    )(parts, deg3, xin, wl, wr, b)


def kernel(x, edge_index, W1l, b1l, W1r, W2l, b2l, W2r):
    src = edge_index[0].astype(jnp.int32)
    dst = edge_index[1].astype(jnp.int32)
    packed = src | (dst << _SHIFT)
    zf = jnp.zeros((NPAD, D), jnp.float32)
    zd = jnp.zeros((NPAD,), jnp.float32)

    part1, deg = _sc_agg_deg(x, packed, zf, zd)
    deg3 = deg.reshape(NC, NPAD, 1)
    h = _dense(part1, deg3, x, W1l, W1r, b1l.reshape(1, D), relu=True)
    part2, _ = _sc_agg_nodeg(h, packed, zf, zd)
    out = _dense(part2, deg3, h, W2l, W2r, b2l.reshape(1, D), relu=False)
    return out
